# baseline (speedup 1.0000x reference)
import jax
import jax.numpy as jnp
from jax.experimental import pallas as pl
from jax.experimental.pallas import tpu as pltpu


def _gcn_body(x_ref, w_ref, b_ref, adj_ref, out_ref, support_ref):
    @pl.when(pl.program_id(0) == 0)
    def _():
        xb = x_ref[...].astype(jnp.bfloat16)
        wb = w_ref[...].astype(jnp.bfloat16)
        s = jnp.dot(xb, wb, preferred_element_type=jnp.float32)
        support_ref[...] = s.astype(jnp.bfloat16)

    a = adj_ref[...].astype(jnp.bfloat16)
    acc = jnp.dot(a, support_ref[...], preferred_element_type=jnp.float32)
    out_ref[...] = acc + b_ref[...]


def kernel(input, adj, W, b):
    N, d_in = input.shape
    d_out = W.shape[1]
    BM = 448

    b2 = b.reshape(1, d_out).astype(jnp.float32)

    return pl.pallas_call(
        _gcn_body,
        grid=(pl.cdiv(N, BM),),
        in_specs=[
            pl.BlockSpec((N, d_in), lambda i: (0, 0)),
            pl.BlockSpec((d_in, d_out), lambda i: (0, 0)),
            pl.BlockSpec((1, d_out), lambda i: (0, 0)),
            pl.BlockSpec((BM, N), lambda i: (i, 0)),
        ],
        out_specs=pl.BlockSpec((BM, d_out), lambda i: (i, 0)),
        out_shape=jax.ShapeDtypeStruct((N, d_out), jnp.float32),
        scratch_shapes=[pltpu.VMEM((N, d_out), jnp.bfloat16)],
    )(input.astype(jnp.float32), W.astype(jnp.float32), b2, adj.astype(jnp.float32))


# speedup vs baseline: 1.0129x; 1.0129x over previous
import jax
import jax.numpy as jnp
from jax.experimental import pallas as pl
from jax.experimental.pallas import tpu as pltpu


def _gcn_body(x_ref, w_ref, b_ref, adj_ref, out_ref, support_ref):
    @pl.when(pl.program_id(0) == 0)
    def _():
        support_ref[...] = jnp.dot(x_ref[...], w_ref[...],
                                   preferred_element_type=jnp.float32)

    acc = jnp.dot(adj_ref[...], support_ref[...],
                  preferred_element_type=jnp.float32)
    out_ref[...] = acc + b_ref[...]


def kernel(input, adj, W, b):
    N, d_in = input.shape
    d_out = W.shape[1]
    BM = 400

    b2 = b.reshape(1, d_out).astype(jnp.float32)

    return pl.pallas_call(
        _gcn_body,
        grid=(N // BM,),
        in_specs=[
            pl.BlockSpec((N, d_in), lambda i: (0, 0)),
            pl.BlockSpec((d_in, d_out), lambda i: (0, 0)),
            pl.BlockSpec((1, d_out), lambda i: (0, 0)),
            pl.BlockSpec((BM, N), lambda i: (i, 0)),
        ],
        out_specs=pl.BlockSpec((BM, d_out), lambda i: (i, 0)),
        out_shape=jax.ShapeDtypeStruct((N, d_out), jnp.float32),
        scratch_shapes=[pltpu.VMEM((N, d_out), jnp.float32)],
    )(input.astype(jnp.float32), W.astype(jnp.float32), b2, adj.astype(jnp.float32))
